# baseline (speedup 1.0000x reference)
import jax
import jax.numpy as jnp
from jax.experimental import pallas as pl
from jax.experimental.pallas import tpu as pltpu
from jax.experimental.pallas import tpu_sc as plsc


_D = 128
_MAXP = 90
_W = 128
_LANES = 16


def kernel(cumulative_positions, embedding):
    b, t = cumulative_positions.shape
    n = b * t
    idx_flat = cumulative_positions.reshape(1, n).astype(jnp.int32)

    mesh = plsc.VectorSubcoreMesh(
        core_axis_name="core", subcore_axis_name="subcore"
    )

    @pl.kernel(
        out_type=jax.ShapeDtypeStruct((n, _D), jnp.float32),
        mesh=mesh,
    )
    def gather_kernel(table_hbm, i_hbm, o_hbm):
        def body(i_vmem, o_vmem):
            @pl.loop(0, _W, step=_LANES)
            def _(c):
                v = i_vmem[0, pl.ds(c, _LANES)]
                v = jnp.minimum(jnp.maximum(v, 0), _MAXP - 1)
                i_vmem[0, pl.ds(c, _LANES)] = v

            pltpu.sync_copy(table_hbm.at[i_vmem.at[0]], o_vmem)

        pltpu.emit_pipeline(
            body,
            grid=(n // _W,),
            in_specs=[pl.BlockSpec((1, _W), index_map=lambda i: (0, i))],
            out_specs=[pl.BlockSpec((_W, _D), index_map=lambda i: (i, 0))],
            core_axis_name=("core", "subcore"),
            dimension_semantics=(pltpu.PARALLEL,),
        )(i_hbm, o_hbm)

    out = gather_kernel(embedding, idx_flat)
    return out.reshape(b, t, _D)


# speedup vs baseline: 1.0004x; 1.0004x over previous
import jax
import jax.numpy as jnp
from jax.experimental import pallas as pl
from jax.experimental.pallas import tpu as pltpu
from jax.experimental.pallas import tpu_sc as plsc


_D = 128
_MAXP = 90
_W = 128
_LANES = 16


def kernel(cumulative_positions, embedding):
    b, t = cumulative_positions.shape
    n = b * t
    idx_flat = cumulative_positions.reshape(1, n).astype(jnp.int32)

    mesh = plsc.VectorSubcoreMesh(
        core_axis_name="core", subcore_axis_name="subcore"
    )

    num_workers = mesh.num_cores * mesh.num_subcores
    steps = n // _W // num_workers

    @pl.kernel(
        out_type=jax.ShapeDtypeStruct((n, _D), jnp.float32),
        mesh=mesh,
    )
    def gather_kernel(table_hbm, i_hbm, o_hbm):
        wid = (
            jax.lax.axis_index("subcore") * mesh.num_cores
            + jax.lax.axis_index("core")
        )
        base = wid * steps

        def body(i_vmem, o_vmem):
            @pl.loop(0, _W, step=_LANES)
            def _(c):
                v = i_vmem[0, pl.ds(c, _LANES)]
                v = jnp.minimum(jnp.maximum(v, 0), _MAXP - 1)
                i_vmem[0, pl.ds(c, _LANES)] = v

            pltpu.sync_copy(table_hbm.at[i_vmem.at[0]], o_vmem)

        pltpu.emit_pipeline(
            body,
            grid=(steps,),
            in_specs=[pl.BlockSpec((1, _W), index_map=lambda i: (0, base + i))],
            out_specs=[
                pl.BlockSpec((_W, _D), index_map=lambda i: (base + i, 0))
            ],
        )(i_hbm, o_hbm)

    out = gather_kernel(embedding, idx_flat)
    return out.reshape(b, t, _D)


# speedup vs baseline: 10.6986x; 10.6942x over previous
import jax
import jax.numpy as jnp
from jax.experimental import pallas as pl
from jax.experimental.pallas import tpu as pltpu
from jax.experimental.pallas import tpu_sc as plsc


_D = 128
_MAXP = 90
_LANES = 16
_CH = 128


def kernel(cumulative_positions, embedding):
    b, t = cumulative_positions.shape
    n = b * t

    mesh = plsc.VectorSubcoreMesh(
        core_axis_name="core", subcore_axis_name="subcore"
    )
    num_workers = mesh.num_cores * mesh.num_subcores
    per_worker = n // num_workers
    nch = per_worker // _CH

    idx3 = cumulative_positions.reshape(num_workers, nch, _CH).astype(
        jnp.int32
    )

    @pl.kernel(
        out_type=jax.ShapeDtypeStruct((n, _D), jnp.float32),
        mesh=mesh,
        scratch_types=[
            pltpu.VMEM((_MAXP, _D), jnp.float32),
            pltpu.VMEM((_CH, _D), jnp.float32),
            pltpu.VMEM((_CH, _D), jnp.float32),
            pltpu.VMEM((_CH,), jnp.int32),
            pltpu.VMEM((_CH,), jnp.int32),
            pltpu.SemaphoreType.DMA,
            pltpu.SemaphoreType.DMA,
            pltpu.SemaphoreType.DMA,
            pltpu.SemaphoreType.DMA,
        ],
    )
    def gather_kernel(
        table_hbm, i_hbm, o_hbm,
        table_v, out0, out1, idx0, idx1,
        isem0, isem1, osem0, osem1,
    ):
        wid = (
            jax.lax.axis_index("subcore") * mesh.num_cores
            + jax.lax.axis_index("core")
        )
        row_base = wid * per_worker

        pltpu.sync_copy(table_hbm, table_v)

        outs = (out0, out1)
        idxs = (idx0, idx1)
        isems = (isem0, isem1)
        osems = (osem0, osem1)

        pltpu.async_copy(i_hbm.at[wid, 0], idx0, isem0)
        pltpu.async_copy(i_hbm.at[wid, 1], idx1, isem1)

        @pl.loop(0, nch, step=2)
        def _(c):
            for bsel in range(2):
                cb = c + bsel
                out_v = outs[bsel]
                idx_s = idxs[bsel]
                isem = isems[bsel]
                osem = osems[bsel]

                pltpu.make_async_copy(i_hbm.at[wid, cb], idx_s, isem).wait()

                @pl.when(cb >= 2)
                def _():
                    pltpu.make_async_copy(
                        out_v,
                        o_hbm.at[pl.ds(row_base + (cb - 2) * _CH, _CH)],
                        osem,
                    ).wait()

                @pl.loop(0, _CH, step=_LANES)
                def _(r0):
                    iv = idx_s[pl.ds(r0, _LANES)]
                    iv = jnp.minimum(jnp.maximum(iv, 0), _MAXP - 1)
                    for l in range(_LANES):
                        i = iv[l]
                        for q in range(_D // _LANES):
                            sl = pl.ds(q * _LANES, _LANES)
                            out_v[r0 + l, sl] = table_v[i, sl]

                pltpu.async_copy(
                    out_v, o_hbm.at[pl.ds(row_base + cb * _CH, _CH)], osem
                )

                @pl.when(cb + 2 < nch)
                def _():
                    pltpu.async_copy(i_hbm.at[wid, cb + 2], idx_s, isem)

        pltpu.make_async_copy(
            out0, o_hbm.at[pl.ds(row_base + (nch - 2) * _CH, _CH)], osem0
        ).wait()
        pltpu.make_async_copy(
            out1, o_hbm.at[pl.ds(row_base + (nch - 1) * _CH, _CH)], osem1
        ).wait()

    out = gather_kernel(embedding, idx3)
    return out.reshape(b, t, _D)


# speedup vs baseline: 33.0956x; 3.0934x over previous
import jax
import jax.numpy as jnp
from jax.experimental import pallas as pl
from jax.experimental.pallas import tpu as pltpu
from jax.experimental.pallas import tpu_sc as plsc


_D = 128
_MAXP = 90
_LANES = 16
_CH = 80
_NBUF = 4


def kernel(cumulative_positions, embedding):
    b, t = cumulative_positions.shape
    n = b * t

    mesh = plsc.VectorSubcoreMesh(
        core_axis_name="core", subcore_axis_name="subcore"
    )
    num_workers = mesh.num_cores * mesh.num_subcores
    per_worker = n // num_workers
    nch = per_worker // _CH

    idx3 = cumulative_positions.reshape(num_workers, nch, _CH).astype(
        jnp.int32
    )

    @pl.kernel(
        out_type=jax.ShapeDtypeStruct((n, _D), jnp.float32),
        mesh=mesh,
        scratch_types=[
            pltpu.VMEM_SHARED((_MAXP, _D), jnp.float32),
            pltpu.VMEM((_NBUF, _CH, _D), jnp.float32),
            pltpu.VMEM((_NBUF, _CH), jnp.int32),
            pltpu.SemaphoreType.DMA((_NBUF,)),
            pltpu.SemaphoreType.DMA((_NBUF,)),
            pltpu.SemaphoreType.DMA((_NBUF,)),
        ],
    )
    def gather_kernel(
        table_hbm, i_hbm, o_hbm,
        table_s, out_v, idx_v, isem, gsem, osem,
    ):
        sid = jax.lax.axis_index("subcore")
        wid = sid * mesh.num_cores + jax.lax.axis_index("core")
        row_base = wid * per_worker

        @pl.when(sid == 0)
        def _():
            pltpu.sync_copy(table_hbm, table_s)

        plsc.subcore_barrier()

        for u in range(_NBUF):
            pltpu.async_copy(i_hbm.at[wid, u], idx_v.at[u], isem.at[u])

        @pl.loop(0, nch, step=_NBUF)
        def _(c0):
            for u in range(_NBUF):
                c = c0 + u
                ob = out_v.at[u]
                ib = idx_v.at[u]

                pltpu.make_async_copy(i_hbm.at[wid, c], ib, isem.at[u]).wait()

                for s in range(_CH // _LANES):
                    sl = pl.ds(s * _LANES, _LANES)
                    ib[sl] = jnp.minimum(jnp.maximum(ib[sl], 0), _MAXP - 1)

                @pl.when(c >= _NBUF)
                def _():
                    pltpu.make_async_copy(
                        ob,
                        o_hbm.at[pl.ds(row_base + (c - _NBUF) * _CH, _CH)],
                        osem.at[u],
                    ).wait()

                pltpu.async_copy(table_s.at[ib], ob, gsem.at[u])

                up = (u - 1) % _NBUF

                @pl.when(c >= 1)
                def _():
                    pltpu.make_async_copy(
                        table_s.at[idx_v.at[up]], out_v.at[up], gsem.at[up]
                    ).wait()
                    pltpu.async_copy(
                        out_v.at[up],
                        o_hbm.at[pl.ds(row_base + (c - 1) * _CH, _CH)],
                        osem.at[up],
                    )

                    @pl.when(c - 1 + _NBUF < nch)
                    def _():
                        pltpu.async_copy(
                            i_hbm.at[wid, c - 1 + _NBUF],
                            idx_v.at[up],
                            isem.at[up],
                        )

        ul = (nch - 1) % _NBUF
        pltpu.make_async_copy(
            table_s.at[idx_v.at[ul]], out_v.at[ul], gsem.at[ul]
        ).wait()
        pltpu.async_copy(
            out_v.at[ul],
            o_hbm.at[pl.ds(row_base + (nch - 1) * _CH, _CH)],
            osem.at[ul],
        )
        for u in range(_NBUF):
            cc = nch - _NBUF + u
            pltpu.make_async_copy(
                out_v.at[u],
                o_hbm.at[pl.ds(row_base + cc * _CH, _CH)],
                osem.at[u],
            ).wait()

    out = gather_kernel(embedding, idx3)
    return out.reshape(b, t, _D)
